# baseline (speedup 1.0000x reference)
import functools

import jax
import jax.numpy as jnp
from jax import lax
from jax.experimental import pallas as pl
from jax.experimental.pallas import tpu as pltpu
from jax.experimental.pallas import tpu_sc as plsc

_B = 16
_NW = 32
_L = 16
_NS = 6144
_NEG = -3.0e38


def _lane_allsum(x):
    idx = lax.iota(jnp.int32, _L)
    for sh in (1, 2, 4, 8):
        x = x + x.at[idx ^ sh].get(mode="promise_in_bounds")
    return x


def _sc_worker(flat_hbm, bnd_hbm, wg_hbm, wout_hbm,
               p_hbm, s_hbm, m_hbm,
               buf0, buf1, wg_v, wout_v, A, logv, mv, sv, bnd_v, pout, esm,
               sem0, sem1,
               *, N0, RW, C, D):
    KD = D // _L
    NCH = RW // C
    wid = lax.axis_index("s") * 2 + lax.axis_index("c")

    pltpu.sync_copy(wg_hbm, wg_v)
    pltpu.sync_copy(wout_hbm, wout_v)
    pltpu.sync_copy(bnd_hbm.at[wid], bnd_v)

    zeros16 = jnp.zeros((_L,), jnp.float32)
    negv = jnp.full((_L,), _NEG, jnp.float32)

    @plsc.parallel_loop(0, (_B * D) // _L, unroll=8)
    def _zero_A(k):
        A[pl.ds(k * _L, _L)] = zeros16
    for b in range(_B):
        mv[pl.ds(b * _L, _L)] = negv
        sv[pl.ds(b * _L, _L)] = zeros16

    lo_vec = bnd_v[pl.ds(0, _L)]
    hi_vec = bnd_v[pl.ds(_L, _L)]

    def process_chunk(buf, r0):
        @plsc.parallel_loop(0, C // 8)
        def row_logit(g):
            r8 = g * 8

            @plsc.parallel_loop(0, KD, unroll=2, carry=(zeros16,) * 8)
            def dotk(k, accs):
                wgk = wg_v[pl.ds(k * _L, _L)]
                return tuple(
                    accs[i] + buf[r8 + i, pl.ds(k * _L, _L)] * wgk
                    for i in range(8))
            for i in range(8):
                logv[pl.ds((r8 + i) * _L, _L)] = _lane_allsum(dotk[i])

        def seg_body(b, carry):
            sel = lax.iota(jnp.int32, _L) == b
            lo_spl = _lane_allsum(jnp.where(sel, lo_vec.astype(jnp.float32),
                                            0.0))
            hi_spl = _lane_allsum(jnp.where(sel, hi_vec.astype(jnp.float32),
                                            0.0))
            lo = jnp.maximum(lo_spl[0].astype(jnp.int32) - r0, 0)
            hi = jnp.minimum(hi_spl[0].astype(jnp.int32) - r0, C)

            @pl.when(hi > lo)
            def _seg():
                @plsc.parallel_loop(0, C, unroll=8, carry=negv)
                def mrow(r, m_acc):
                    in_seg = (r >= lo) & (r < hi)
                    return jnp.maximum(
                        m_acc,
                        jnp.where(in_seg, logv[pl.ds(r * _L, _L)], negv))
                m_cb = mrow

                m_old = mv[pl.ds(b * _L, _L)]
                m_new = jnp.maximum(m_old, m_cb)
                alpha = jnp.exp(jnp.maximum(m_old - m_new,
                                            jnp.float32(-100.0)))
                sv[pl.ds(b * _L, _L)] = sv[pl.ds(b * _L, _L)] * alpha
                mv[pl.ds(b * _L, _L)] = m_new

                @plsc.parallel_loop(0, KD, unroll=8)
                def resc(k):
                    A[pl.ds(b * D + k * _L, _L)] = (
                        A[pl.ds(b * D + k * _L, _L)] * alpha)

                @plsc.parallel_loop(0, C, unroll=4, carry=zeros16)
                def erow(r, s_acc):
                    ev = jnp.where(
                        (r >= lo) & (r < hi),
                        jnp.exp(jnp.maximum(logv[pl.ds(r * _L, _L)] - m_new,
                                            jnp.float32(-100.0))),
                        zeros16)
                    esm[r] = ev[0]
                    return s_acc + ev
                sv[pl.ds(b * _L, _L)] = sv[pl.ds(b * _L, _L)] + erow

                @plsc.parallel_loop(0, KD, unroll=2)
                def colk(k):
                    a0 = A[pl.ds(b * D + k * _L, _L)]

                    @plsc.parallel_loop(0, C // 4, unroll=2,
                                        carry=(a0, zeros16, zeros16, zeros16))
                    def rowin(q, accs):
                        r4 = q * 4
                        return tuple(
                            accs[i]
                            + esm[r4 + i] * buf[r4 + i, pl.ds(k * _L, _L)]
                            for i in range(4))
                    a = rowin
                    A[pl.ds(b * D + k * _L, _L)] = (
                        (a[0] + a[1]) + (a[2] + a[3]))
            return carry
        lax.fori_loop(0, _B, seg_body, 0)

    base = N0 + wid * RW
    bufs = (buf0, buf1)
    sems = (sem0, sem1)
    handles = [
        pltpu.async_copy(flat_hbm.at[pl.ds(base + c * C, C)],
                         bufs[c % 2], sems[c % 2])
        for c in range(min(2, NCH))]
    for c in range(NCH):
        handles[c].wait()
        process_chunk(bufs[c % 2], base + c * C)
        if c + 2 < NCH:
            handles.append(
                pltpu.async_copy(flat_hbm.at[pl.ds(base + (c + 2) * C, C)],
                                 bufs[c % 2], sems[c % 2]))

    for b in range(_B):
        for j in range(2):
            @plsc.parallel_loop(0, KD, unroll=8, carry=zeros16)
            def doto(k, acc, b=b, j=j):
                return acc + (A[pl.ds(b * D + k * _L, _L)]
                              * wout_v[pl.ds(j * D + k * _L, _L)])
            pout[pl.ds((b * 2 + j) * _L, _L)] = doto

    pltpu.sync_copy(pout, p_hbm.at[wid])
    pltpu.sync_copy(sv, s_hbm.at[wid])
    pltpu.sync_copy(mv, m_hbm.at[wid])


def _tc_flash_body(flat_ref, segs_ref, wg_ref, wo_ref,
                   m_out, s_out, p_out, A, m, s, *, T, G):
    i = pl.program_id(0)

    @pl.when(i == 0)
    def _init():
        A[...] = jnp.zeros_like(A)
        m[...] = jnp.full_like(m, _NEG)
        s[...] = jnp.zeros_like(s)

    flat_c = flat_ref[...]
    segs2 = segs_ref[0]
    onehot = (lax.broadcasted_iota(jnp.int32, (_B, T), 0) == segs2)

    logits_t = lax.dot_general(
        wg_ref[...], flat_c, (((0,), (1,)), ((), ())),
        preferred_element_type=jnp.float32)

    negc = jnp.float32(_NEG)
    masked = jnp.where(onehot, logits_t, negc)
    m_c = jnp.max(masked, axis=1, keepdims=True)
    m_old = m[...]
    m_new = jnp.maximum(m_old, m_c)
    alpha = jnp.exp(jnp.maximum(m_old - m_new, jnp.float32(-100.0)))

    m_sel = jnp.max(jnp.where(onehot, m_new, negc), axis=0, keepdims=True)
    e_t = jnp.exp(jnp.maximum(logits_t - m_sel, jnp.float32(-100.0)))
    we = jnp.where(onehot, e_t, 0.0)
    s_c = jnp.sum(we, axis=1, keepdims=True)
    A_c = jnp.dot(we, flat_c, preferred_element_type=jnp.float32)

    A[...] = A[...] * alpha + A_c
    s[...] = s[...] * alpha + s_c
    m[...] = m_new

    @pl.when(i == G - 1)
    def _fin():
        m_out[...] = m[...]
        s_out[...] = s[...]
        p_out[...] = jnp.dot(A[...], wo_ref[...],
                             preferred_element_type=jnp.float32)


def _combine_body(p_ref, s_ref, m_ref, mt_ref, st_ref, pt_ref, bo_ref,
                  out_ref):
    m_w = jnp.max(m_ref[...].reshape(_NW, _B, _L), axis=2)
    s_w = jnp.max(s_ref[...].reshape(_NW, _B, _L), axis=2)
    p_w = jnp.sum(p_ref[...].reshape(_NW, _B, 2, _L), axis=3)
    m_t = mt_ref[...][:, 0]
    s_t = st_ref[...][:, 0]
    p_t = pt_ref[...]

    M = jnp.maximum(jnp.max(m_w, axis=0), m_t)
    sc_scale = jnp.exp(jnp.maximum(m_w - M[None, :], jnp.float32(-100.0)))
    tc_scale = jnp.exp(jnp.maximum(m_t - M, jnp.float32(-100.0)))
    S = jnp.sum(s_w * sc_scale, axis=0) + s_t * tc_scale
    P = (jnp.sum(p_w * sc_scale[:, :, None], axis=0)
         + p_t * tc_scale[:, None])
    denom = (S + 1e-16)[:, None]
    out_ref[...] = P / denom + bo_ref[...] * (S[:, None] / denom)


def kernel(flat, segment_ids, W_gate, b_gate, W_out, b_out):
    N, D = flat.shape
    N0 = N - _NS
    RW = _NS // _NW
    C = 32
    segs32 = segment_ids.astype(jnp.int32)
    ids = jnp.arange(_B, dtype=jnp.int32)
    starts = jnp.sum((segs32[:, None] < ids[None, :]).astype(jnp.int32),
                     axis=0)
    ends = jnp.sum((segs32[:, None] <= ids[None, :]).astype(jnp.int32),
                   axis=0)
    wlo = N0 + jnp.arange(_NW, dtype=jnp.int32)[:, None] * RW
    whi = wlo + RW
    lo = jnp.clip(starts[None, :], wlo, whi)
    hi = jnp.clip(ends[None, :], wlo, whi)
    bounds = jnp.concatenate([lo, hi], axis=1)

    mesh = plsc.VectorSubcoreMesh(core_axis_name="c", subcore_axis_name="s")
    sc = pl.kernel(
        functools.partial(_sc_worker, N0=N0, RW=RW, C=C, D=D),
        mesh=mesh,
        out_type=(
            jax.ShapeDtypeStruct((_NW, _B * 2 * _L), jnp.float32),
            jax.ShapeDtypeStruct((_NW, _B * _L), jnp.float32),
            jax.ShapeDtypeStruct((_NW, _B * _L), jnp.float32),
        ),
        scratch_types=[
            pltpu.VMEM((C, D), jnp.float32),
            pltpu.VMEM((C, D), jnp.float32),
            pltpu.VMEM((D,), jnp.float32),
            pltpu.VMEM((2 * D,), jnp.float32),
            pltpu.VMEM((_B * D,), jnp.float32),
            pltpu.VMEM((C * _L,), jnp.float32),
            pltpu.VMEM((_B * _L,), jnp.float32),
            pltpu.VMEM((_B * _L,), jnp.float32),
            pltpu.VMEM((2 * _B,), jnp.int32),
            pltpu.VMEM((_B * 2 * _L,), jnp.float32),
            pltpu.SMEM((C,), jnp.float32),
            pltpu.SemaphoreType.DMA,
            pltpu.SemaphoreType.DMA,
        ],
    )
    P, S, M = sc(flat, bounds, W_gate.reshape(D), W_out.T.reshape(2 * D))

    T = 2048
    G = N0 // T
    segs_tc = segs32.reshape(N // T, 1, T)
    m_t, s_t, p_t = pl.pallas_call(
        functools.partial(_tc_flash_body, T=T, G=G),
        grid=(G,),
        in_specs=[
            pl.BlockSpec((T, D), lambda i: (i, 0)),
            pl.BlockSpec((1, 1, T), lambda i: (i, 0, 0)),
            pl.BlockSpec((D, 1), lambda i: (0, 0)),
            pl.BlockSpec((D, 2), lambda i: (0, 0)),
        ],
        out_specs=(
            pl.BlockSpec((_B, 1), lambda i: (0, 0)),
            pl.BlockSpec((_B, 1), lambda i: (0, 0)),
            pl.BlockSpec((_B, 2), lambda i: (0, 0)),
        ),
        out_shape=(
            jax.ShapeDtypeStruct((_B, 1), jnp.float32),
            jax.ShapeDtypeStruct((_B, 1), jnp.float32),
            jax.ShapeDtypeStruct((_B, 2), jnp.float32),
        ),
        scratch_shapes=[
            pltpu.VMEM((_B, D), jnp.float32),
            pltpu.VMEM((_B, 1), jnp.float32),
            pltpu.VMEM((_B, 1), jnp.float32),
        ],
        compiler_params=pltpu.CompilerParams(
            dimension_semantics=("arbitrary",)),
    )(flat, segs_tc, W_gate, W_out)

    return pl.pallas_call(
        _combine_body,
        out_shape=jax.ShapeDtypeStruct((_B, 2), jnp.float32),
    )(P, S, M, m_t, s_t, p_t, b_out.reshape(1, 2))


# speedup vs baseline: 1.1099x; 1.1099x over previous
import functools

import jax
import jax.numpy as jnp
from jax import lax
from jax.experimental import pallas as pl
from jax.experimental.pallas import tpu as pltpu
from jax.experimental.pallas import tpu_sc as plsc

_B = 16
_NW = 32
_L = 16
_NS = 4096
_NEG = -3.0e38


def _lane_allsum(x):
    idx = lax.iota(jnp.int32, _L)
    for sh in (1, 2, 4, 8):
        x = x + x.at[idx ^ sh].get(mode="promise_in_bounds")
    return x


def _sc_worker(flat_hbm, bnd_hbm, wg_hbm, wout_hbm,
               p_hbm, s_hbm, m_hbm,
               buf0, buf1, wg_v, wout_v, A, logv, mv, sv, bnd_v, pout, esm,
               sem0, sem1,
               *, N0, RW, C, D):
    KD = D // _L
    NCH = RW // C
    wid = lax.axis_index("s") * 2 + lax.axis_index("c")

    pltpu.sync_copy(wg_hbm, wg_v)
    pltpu.sync_copy(wout_hbm, wout_v)
    pltpu.sync_copy(bnd_hbm.at[wid], bnd_v)

    zeros16 = jnp.zeros((_L,), jnp.float32)
    negv = jnp.full((_L,), _NEG, jnp.float32)

    @plsc.parallel_loop(0, (_B * D) // _L, unroll=8)
    def _zero_A(k):
        A[pl.ds(k * _L, _L)] = zeros16
    for b in range(_B):
        mv[pl.ds(b * _L, _L)] = negv
        sv[pl.ds(b * _L, _L)] = zeros16

    lo_vec = bnd_v[pl.ds(0, _L)]
    hi_vec = bnd_v[pl.ds(_L, _L)]

    def process_chunk(buf, r0):
        @plsc.parallel_loop(0, C // 8)
        def row_logit(g):
            r8 = g * 8

            @plsc.parallel_loop(0, KD, unroll=2, carry=(zeros16,) * 8)
            def dotk(k, accs):
                wgk = wg_v[pl.ds(k * _L, _L)]
                return tuple(
                    accs[i] + buf[r8 + i, pl.ds(k * _L, _L)] * wgk
                    for i in range(8))
            for i in range(8):
                logv[pl.ds((r8 + i) * _L, _L)] = _lane_allsum(dotk[i])

        def seg_body(b, carry):
            sel = lax.iota(jnp.int32, _L) == b
            lo_spl = _lane_allsum(jnp.where(sel, lo_vec.astype(jnp.float32),
                                            0.0))
            hi_spl = _lane_allsum(jnp.where(sel, hi_vec.astype(jnp.float32),
                                            0.0))
            lo = jnp.maximum(lo_spl[0].astype(jnp.int32) - r0, 0)
            hi = jnp.minimum(hi_spl[0].astype(jnp.int32) - r0, C)

            @pl.when(hi > lo)
            def _seg():
                @plsc.parallel_loop(0, C, unroll=8, carry=negv)
                def mrow(r, m_acc):
                    in_seg = (r >= lo) & (r < hi)
                    return jnp.maximum(
                        m_acc,
                        jnp.where(in_seg, logv[pl.ds(r * _L, _L)], negv))
                m_cb = mrow

                m_old = mv[pl.ds(b * _L, _L)]
                m_new = jnp.maximum(m_old, m_cb)
                alpha = jnp.exp(jnp.maximum(m_old - m_new,
                                            jnp.float32(-100.0)))
                sv[pl.ds(b * _L, _L)] = sv[pl.ds(b * _L, _L)] * alpha
                mv[pl.ds(b * _L, _L)] = m_new

                @plsc.parallel_loop(0, KD, unroll=8)
                def resc(k):
                    A[pl.ds(b * D + k * _L, _L)] = (
                        A[pl.ds(b * D + k * _L, _L)] * alpha)

                @plsc.parallel_loop(0, C, unroll=4, carry=zeros16)
                def erow(r, s_acc):
                    ev = jnp.where(
                        (r >= lo) & (r < hi),
                        jnp.exp(jnp.maximum(logv[pl.ds(r * _L, _L)] - m_new,
                                            jnp.float32(-100.0))),
                        zeros16)
                    esm[r] = ev[0]
                    return s_acc + ev
                sv[pl.ds(b * _L, _L)] = sv[pl.ds(b * _L, _L)] + erow

                @plsc.parallel_loop(0, KD, unroll=2)
                def colk(k):
                    a0 = A[pl.ds(b * D + k * _L, _L)]

                    @plsc.parallel_loop(0, C // 4, unroll=2,
                                        carry=(a0, zeros16, zeros16, zeros16))
                    def rowin(q, accs):
                        r4 = q * 4
                        return tuple(
                            accs[i]
                            + esm[r4 + i] * buf[r4 + i, pl.ds(k * _L, _L)]
                            for i in range(4))
                    a = rowin
                    A[pl.ds(b * D + k * _L, _L)] = (
                        (a[0] + a[1]) + (a[2] + a[3]))
            return carry
        lax.fori_loop(0, _B, seg_body, 0)

    base = N0 + wid * RW
    bufs = (buf0, buf1)
    sems = (sem0, sem1)
    handles = [
        pltpu.async_copy(flat_hbm.at[pl.ds(base + c * C, C)],
                         bufs[c % 2], sems[c % 2])
        for c in range(min(2, NCH))]
    for c in range(NCH):
        handles[c].wait()
        process_chunk(bufs[c % 2], base + c * C)
        if c + 2 < NCH:
            handles.append(
                pltpu.async_copy(flat_hbm.at[pl.ds(base + (c + 2) * C, C)],
                                 bufs[c % 2], sems[c % 2]))

    for b in range(_B):
        for j in range(2):
            @plsc.parallel_loop(0, KD, unroll=8, carry=zeros16)
            def doto(k, acc, b=b, j=j):
                return acc + (A[pl.ds(b * D + k * _L, _L)]
                              * wout_v[pl.ds(j * D + k * _L, _L)])
            pout[pl.ds((b * 2 + j) * _L, _L)] = doto

    pltpu.sync_copy(pout, p_hbm.at[wid])
    pltpu.sync_copy(sv, s_hbm.at[wid])
    pltpu.sync_copy(mv, m_hbm.at[wid])


def _tc_flash_body(flat_ref, segs_ref, wg_ref, wo_ref,
                   m_out, s_out, p_out, A, m, s, *, T, G):
    i = pl.program_id(0)

    @pl.when(i == 0)
    def _init():
        A[...] = jnp.zeros_like(A)
        m[...] = jnp.full_like(m, _NEG)
        s[...] = jnp.zeros_like(s)

    flat_c = flat_ref[...]
    segs2 = segs_ref[0]
    onehot = (lax.broadcasted_iota(jnp.int32, (_B, T), 0) == segs2)

    logits_t = lax.dot_general(
        wg_ref[...], flat_c, (((0,), (1,)), ((), ())),
        preferred_element_type=jnp.float32)

    negc = jnp.float32(_NEG)
    masked = jnp.where(onehot, logits_t, negc)
    m_c = jnp.max(masked, axis=1, keepdims=True)
    m_old = m[...]
    m_new = jnp.maximum(m_old, m_c)
    alpha = jnp.exp(jnp.maximum(m_old - m_new, jnp.float32(-100.0)))

    m_sel = jnp.max(jnp.where(onehot, m_new, negc), axis=0, keepdims=True)
    e_t = jnp.exp(jnp.maximum(logits_t - m_sel, jnp.float32(-100.0)))
    we = jnp.where(onehot, e_t, 0.0)
    s_c = jnp.sum(we, axis=1, keepdims=True)
    A_c = jnp.dot(we, flat_c, preferred_element_type=jnp.float32)

    A[...] = A[...] * alpha + A_c
    s[...] = s[...] * alpha + s_c
    m[...] = m_new

    @pl.when(i == G - 1)
    def _fin():
        m_out[...] = m[...]
        s_out[...] = s[...]
        p_out[...] = jnp.dot(A[...], wo_ref[...],
                             preferred_element_type=jnp.float32)


def _combine_body(p_ref, s_ref, m_ref, mt_ref, st_ref, pt_ref, bo_ref,
                  out_ref):
    m_w = jnp.max(m_ref[...].reshape(_NW, _B, _L), axis=2)
    s_w = jnp.max(s_ref[...].reshape(_NW, _B, _L), axis=2)
    p_w = jnp.sum(p_ref[...].reshape(_NW, _B, 2, _L), axis=3)
    m_t = mt_ref[...][:, 0]
    s_t = st_ref[...][:, 0]
    p_t = pt_ref[...]

    M = jnp.maximum(jnp.max(m_w, axis=0), m_t)
    sc_scale = jnp.exp(jnp.maximum(m_w - M[None, :], jnp.float32(-100.0)))
    tc_scale = jnp.exp(jnp.maximum(m_t - M, jnp.float32(-100.0)))
    S = jnp.sum(s_w * sc_scale, axis=0) + s_t * tc_scale
    P = (jnp.sum(p_w * sc_scale[:, :, None], axis=0)
         + p_t * tc_scale[:, None])
    denom = (S + 1e-16)[:, None]
    out_ref[...] = P / denom + bo_ref[...] * (S[:, None] / denom)


def kernel(flat, segment_ids, W_gate, b_gate, W_out, b_out):
    N, D = flat.shape
    N0 = N - _NS
    RW = _NS // _NW
    C = 32
    segs32 = segment_ids.astype(jnp.int32)
    ids = jnp.arange(_B, dtype=jnp.int32)
    starts = jnp.sum((segs32[:, None] < ids[None, :]).astype(jnp.int32),
                     axis=0)
    ends = jnp.sum((segs32[:, None] <= ids[None, :]).astype(jnp.int32),
                   axis=0)
    wlo = N0 + jnp.arange(_NW, dtype=jnp.int32)[:, None] * RW
    whi = wlo + RW
    lo = jnp.clip(starts[None, :], wlo, whi)
    hi = jnp.clip(ends[None, :], wlo, whi)
    bounds = jnp.concatenate([lo, hi], axis=1)

    mesh = plsc.VectorSubcoreMesh(core_axis_name="c", subcore_axis_name="s")
    sc = pl.kernel(
        functools.partial(_sc_worker, N0=N0, RW=RW, C=C, D=D),
        mesh=mesh,
        out_type=(
            jax.ShapeDtypeStruct((_NW, _B * 2 * _L), jnp.float32),
            jax.ShapeDtypeStruct((_NW, _B * _L), jnp.float32),
            jax.ShapeDtypeStruct((_NW, _B * _L), jnp.float32),
        ),
        scratch_types=[
            pltpu.VMEM((C, D), jnp.float32),
            pltpu.VMEM((C, D), jnp.float32),
            pltpu.VMEM((D,), jnp.float32),
            pltpu.VMEM((2 * D,), jnp.float32),
            pltpu.VMEM((_B * D,), jnp.float32),
            pltpu.VMEM((C * _L,), jnp.float32),
            pltpu.VMEM((_B * _L,), jnp.float32),
            pltpu.VMEM((_B * _L,), jnp.float32),
            pltpu.VMEM((2 * _B,), jnp.int32),
            pltpu.VMEM((_B * 2 * _L,), jnp.float32),
            pltpu.SMEM((C,), jnp.float32),
            pltpu.SemaphoreType.DMA,
            pltpu.SemaphoreType.DMA,
        ],
    )
    P, S, M = sc(flat, bounds, W_gate.reshape(D), W_out.T.reshape(2 * D))

    T = 4096
    G = N0 // T
    segs_tc = segs32.reshape(N // T, 1, T)
    m_t, s_t, p_t = pl.pallas_call(
        functools.partial(_tc_flash_body, T=T, G=G),
        grid=(G,),
        in_specs=[
            pl.BlockSpec((T, D), lambda i: (i, 0)),
            pl.BlockSpec((1, 1, T), lambda i: (i, 0, 0)),
            pl.BlockSpec((D, 1), lambda i: (0, 0)),
            pl.BlockSpec((D, 2), lambda i: (0, 0)),
        ],
        out_specs=(
            pl.BlockSpec((_B, 1), lambda i: (0, 0)),
            pl.BlockSpec((_B, 1), lambda i: (0, 0)),
            pl.BlockSpec((_B, 2), lambda i: (0, 0)),
        ),
        out_shape=(
            jax.ShapeDtypeStruct((_B, 1), jnp.float32),
            jax.ShapeDtypeStruct((_B, 1), jnp.float32),
            jax.ShapeDtypeStruct((_B, 2), jnp.float32),
        ),
        scratch_shapes=[
            pltpu.VMEM((_B, D), jnp.float32),
            pltpu.VMEM((_B, 1), jnp.float32),
            pltpu.VMEM((_B, 1), jnp.float32),
        ],
        compiler_params=pltpu.CompilerParams(
            dimension_semantics=("arbitrary",)),
    )(flat, segs_tc, W_gate, W_out)

    return pl.pallas_call(
        _combine_body,
        out_shape=jax.ShapeDtypeStruct((_B, 2), jnp.float32),
    )(P, S, M, m_t, s_t, p_t, b_out.reshape(1, 2))
